# Initial kernel scaffold; baseline (speedup 1.0000x reference)
import functools

import jax
import jax.numpy as jnp
from jax import lax
from jax.experimental import pallas as pl
from jax.experimental.pallas import tpu as pltpu
from jax.experimental.pallas import tpu_sc as plsc

NC = 2
NS = 16
NW = NC * NS
LANES = 16


def _mesh():
    return plsc.VectorSubcoreMesh(core_axis_name="c", subcore_axis_name="s")


@functools.lru_cache(maxsize=None)
def _deg_kernel(E, N):
    e_per_tile = E // NW

    @functools.partial(
        pl.kernel,
        out_type=jax.ShapeDtypeStruct((NW, N), jnp.float32),
        mesh=_mesh(),
        scratch_types=[
            pltpu.VMEM((e_per_tile,), jnp.int32),
            pltpu.VMEM((N,), jnp.float32),
        ],
    )
    def k(dst_hbm, out_hbm, idx_v, deg_v):
        c = lax.axis_index("c")
        s = lax.axis_index("s")
        wid = c * NS + s

        zeros = jnp.zeros((LANES,), jnp.float32)

        def zero_body(i, _):
            deg_v[pl.ds(i * LANES, LANES)] = zeros
            return 0

        lax.fori_loop(0, N // LANES, zero_body, 0)

        pltpu.sync_copy(dst_hbm.at[pl.ds(wid * e_per_tile, e_per_tile)], idx_v)

        ones = jnp.ones((LANES,), jnp.float32)

        def body(i, _):
            idx = idx_v[pl.ds(i * LANES, LANES)]
            plsc.addupdate_scatter(deg_v, [idx], ones)
            return 0

        lax.fori_loop(0, e_per_tile // LANES, body, 0)
        pltpu.sync_copy(deg_v, out_hbm.at[wid])

    return k


@functools.lru_cache(maxsize=None)
def _agg_kernel(E, N, D, CH):
    e_per_sc = E // NC
    e_per_tile = e_per_sc // NS
    n_chunks = e_per_tile // CH
    rows_per_tile = N // NS

    @functools.partial(
        pl.kernel,
        out_type=jax.ShapeDtypeStruct((NC, N, D), jnp.float32),
        mesh=_mesh(),
        scratch_types=[
            pltpu.VMEM((CH,), jnp.int32),
            pltpu.VMEM((CH,), jnp.int32),
            pltpu.VMEM((CH, D), jnp.float32),
            pltpu.VMEM_SHARED((N, D), jnp.float32),
            pltpu.SemaphoreType.DMA,
        ],
    )
    def k(u_hbm, src_hbm, dst_hbm, zeros_hbm, out_hbm, si_v, di_v, rows_v,
          acc_sh, sem):
        c = lax.axis_index("c")
        s = lax.axis_index("s")
        row0 = s * rows_per_tile

        pltpu.sync_copy(zeros_hbm, acc_sh.at[pl.ds(row0, rows_per_tile)])
        plsc.subcore_barrier()

        base = c * e_per_sc + s * e_per_tile

        def body(i, _):
            off = base + i * CH
            pltpu.sync_copy(src_hbm.at[pl.ds(off, CH)], si_v)
            pltpu.sync_copy(dst_hbm.at[pl.ds(off, CH)], di_v)
            pltpu.async_copy(u_hbm.at[si_v], rows_v, sem).wait()
            pltpu.sync_copy(rows_v, acc_sh.at[di_v], add=True)
            return 0

        lax.fori_loop(0, n_chunks, body, 0)
        plsc.subcore_barrier()
        pltpu.sync_copy(acc_sh.at[pl.ds(row0, rows_per_tile)],
                        out_hbm.at[c, pl.ds(row0, rows_per_tile)])

    return k


@functools.lru_cache(maxsize=None)
def _mm_kernel(N, F, H, DP, BLK):
    grid = N // BLK

    def body(s0, s1, u1, dinv, w1, b1, w2, o):
        agg = (s0[...] + s1[...] + u1[...]) * dinv[...]
        h = jnp.dot(agg, w1[...], preferred_element_type=jnp.float32)
        h = jnp.maximum(h + b1[...], 0.0)
        t = jnp.dot(h, w2[...], preferred_element_type=jnp.float32)
        o[...] = t * dinv[...]

    return pl.pallas_call(
        body,
        grid=(grid,),
        in_specs=[
            pl.BlockSpec((BLK, F), lambda i: (i, 0)),
            pl.BlockSpec((BLK, F), lambda i: (i, 0)),
            pl.BlockSpec((BLK, F), lambda i: (i, 0)),
            pl.BlockSpec((BLK, 1), lambda i: (i, 0)),
            pl.BlockSpec((F, H), lambda i: (0, 0)),
            pl.BlockSpec((1, H), lambda i: (0, 0)),
            pl.BlockSpec((H, DP), lambda i: (0, 0)),
        ],
        out_specs=pl.BlockSpec((BLK, DP), lambda i: (i, 0)),
        out_shape=jax.ShapeDtypeStruct((N, DP), jnp.float32),
    )


def kernel(x, edge_index, W1, b1, W2, b2):
    N, F = x.shape
    E = edge_index.shape[1]
    H = W1.shape[1]
    C = W2.shape[1]
    D2 = 16
    DP = 128

    src = edge_index[0]
    dst = edge_index[1]

    degp = _deg_kernel(E, N)(dst)
    deg = jnp.sum(degp, axis=0) + 1.0
    dinv = lax.rsqrt(deg)

    u1 = x * dinv[:, None]
    z1 = jnp.zeros((N // NS, F), jnp.float32)
    S1 = _agg_kernel(E, N, F, 80)(u1, src, dst, z1)

    W2p = jnp.zeros((H, DP), jnp.float32).at[:, :C].set(W2)
    u2p = _mm_kernel(N, F, H, DP, 2000)(
        S1[0], S1[1], u1, dinv[:, None], W1, b1[None, :], W2p)
    u2 = u2p[:, :D2]

    z2 = jnp.zeros((N // NS, D2), jnp.float32)
    S2 = _agg_kernel(E, N, D2, 80)(u2, src, dst, z2)

    out = (S2[0] + S2[1] + u2)[:, :C] * dinv[:, None] + b2[None, :]
    return out


# speedup vs baseline: 17.1441x; 17.1441x over previous
import functools

import jax
import jax.numpy as jnp
from jax import lax
from jax.experimental import pallas as pl
from jax.experimental.pallas import tpu as pltpu
from jax.experimental.pallas import tpu_sc as plsc

NC = 2
NS = 16
NW = NC * NS
LANES = 16


def _mesh():
    return plsc.VectorSubcoreMesh(core_axis_name="c", subcore_axis_name="s")


@functools.lru_cache(maxsize=None)
def _deg_kernel(E, N):
    e_per_tile = E // NW

    @functools.partial(
        pl.kernel,
        out_type=jax.ShapeDtypeStruct((NW * N,), jnp.float32),
        mesh=_mesh(),
        scratch_types=[
            pltpu.VMEM((e_per_tile,), jnp.int32),
            pltpu.VMEM((N,), jnp.float32),
        ],
        compiler_params=pltpu.CompilerParams(needs_layout_passes=False, use_tc_tiling_on_sc=False),
    )
    def k(dst_hbm, out_hbm, idx_v, deg_v):
        c = lax.axis_index("c")
        s = lax.axis_index("s")
        wid = c * NS + s

        zeros = jnp.zeros((LANES,), jnp.float32)

        def zero_body(i, _):
            deg_v[pl.ds(i * LANES, LANES)] = zeros
            return 0

        lax.fori_loop(0, N // LANES, zero_body, 0)

        pltpu.sync_copy(dst_hbm.at[pl.ds(wid * e_per_tile, e_per_tile)], idx_v)

        ones = jnp.ones((LANES,), jnp.float32)

        def body(i, _):
            idx = idx_v[pl.ds(i * LANES, LANES)]
            plsc.addupdate_scatter(deg_v, [idx], ones)
            return 0

        lax.fori_loop(0, e_per_tile // LANES, body, 0)
        pltpu.sync_copy(deg_v, out_hbm.at[pl.ds(wid * N, N)])

    return k


@functools.lru_cache(maxsize=None)
def _agg_kernel(E, N, D, CH):
    e_per_sc = E // NC
    e_per_tile = e_per_sc // NS
    n_chunks = e_per_tile // CH
    RC = 80
    n_row_chunks = N // RC
    row_iters = -(-n_row_chunks // NS)

    @functools.partial(
        pl.kernel,
        out_type=jax.ShapeDtypeStruct((NC, N, D), jnp.float32),
        mesh=_mesh(),
        scratch_types=[
            pltpu.VMEM((CH,), jnp.int32),
            pltpu.VMEM((CH,), jnp.int32),
            pltpu.VMEM((CH, D), jnp.float32),
            pltpu.VMEM_SHARED((N, D), jnp.float32),
            pltpu.SemaphoreType.DMA,
        ],
        compiler_params=pltpu.CompilerParams(needs_layout_passes=False, use_tc_tiling_on_sc=False),
    )
    def k(u_hbm, src_hbm, dst_hbm, zeros_hbm, out_hbm, si_v, di_v, rows_v,
          acc_sh, sem):
        c = lax.axis_index("c")
        s = lax.axis_index("s")

        def zero_body(i, _):
            rc = i * NS + s

            @pl.when(rc < n_row_chunks)
            def _():
                pltpu.sync_copy(zeros_hbm, acc_sh.at[pl.ds(rc * RC, RC)])

            return 0

        lax.fori_loop(0, row_iters, zero_body, 0)
        plsc.subcore_barrier()

        base = c * e_per_sc + s * e_per_tile

        def body(i, _):
            off = base + i * CH
            pltpu.sync_copy(src_hbm.at[pl.ds(off, CH)], si_v)
            pltpu.sync_copy(dst_hbm.at[pl.ds(off, CH)], di_v)
            pltpu.async_copy(u_hbm.at[si_v], rows_v, sem).wait()
            pltpu.sync_copy(rows_v, acc_sh.at[di_v], add=True)
            return 0

        lax.fori_loop(0, n_chunks, body, 0)
        plsc.subcore_barrier()

        def out_body(i, _):
            rc = i * NS + s

            @pl.when(rc < n_row_chunks)
            def _():
                pltpu.sync_copy(acc_sh.at[pl.ds(rc * RC, RC)],
                                out_hbm.at[c, pl.ds(rc * RC, RC)])

            return 0

        lax.fori_loop(0, row_iters, out_body, 0)

    return k


@functools.lru_cache(maxsize=None)
def _mm_kernel(N, F, H, DP, BLK):
    grid = N // BLK

    def body(s0, s1, u1, dinv, w1, b1, w2, o):
        agg = (s0[...] + s1[...] + u1[...]) * dinv[...]
        h = jnp.dot(agg, w1[...], preferred_element_type=jnp.float32)
        h = jnp.maximum(h + b1[...], 0.0)
        t = jnp.dot(h, w2[...], preferred_element_type=jnp.float32)
        o[...] = t * dinv[...]

    return pl.pallas_call(
        body,
        grid=(grid,),
        in_specs=[
            pl.BlockSpec((BLK, F), lambda i: (i, 0)),
            pl.BlockSpec((BLK, F), lambda i: (i, 0)),
            pl.BlockSpec((BLK, F), lambda i: (i, 0)),
            pl.BlockSpec((BLK, 1), lambda i: (i, 0)),
            pl.BlockSpec((F, H), lambda i: (0, 0)),
            pl.BlockSpec((1, H), lambda i: (0, 0)),
            pl.BlockSpec((H, DP), lambda i: (0, 0)),
        ],
        out_specs=pl.BlockSpec((BLK, DP), lambda i: (i, 0)),
        out_shape=jax.ShapeDtypeStruct((N, DP), jnp.float32),
    )


def kernel(x, edge_index, W1, b1, W2, b2):
    N, F = x.shape
    E = edge_index.shape[1]
    H = W1.shape[1]
    C = W2.shape[1]
    D2 = 16
    DP = 128

    src = edge_index[0]
    dst = edge_index[1]

    degp = _deg_kernel(E, N)(dst).reshape(NW, N)
    deg = jnp.sum(degp, axis=0) + 1.0
    dinv = lax.rsqrt(deg)

    u1 = x * dinv[:, None]
    z1 = jnp.zeros((80, F), jnp.float32)
    S1 = _agg_kernel(E, N, F, 80)(u1, src, dst, z1)

    W2p = jnp.zeros((H, DP), jnp.float32).at[:, :C].set(W2)
    u2p = _mm_kernel(N, F, H, DP, 2000)(
        S1[0], S1[1], u1, dinv[:, None], W1, b1[None, :], W2p)
    u2 = u2p[:, :D2]

    z2 = jnp.zeros((80, D2), jnp.float32)
    S2 = _agg_kernel(E, N, D2, 80)(u2, src, dst, z2)

    out = (S2[0] + S2[1] + u2)[:, :C] * dinv[:, None] + b2[None, :]
    return out


# speedup vs baseline: 40.7090x; 2.3745x over previous
import functools

import jax
import jax.numpy as jnp
from jax import lax
from jax.experimental import pallas as pl
from jax.experimental.pallas import tpu as pltpu
from jax.experimental.pallas import tpu_sc as plsc

NC = 2
NS = 16
NW = NC * NS
LANES = 16


def _mesh():
    return plsc.VectorSubcoreMesh(core_axis_name="c", subcore_axis_name="s")


@functools.lru_cache(maxsize=None)
def _deg_kernel(E, N):
    e_per_tile = E // NW

    @functools.partial(
        pl.kernel,
        out_type=jax.ShapeDtypeStruct((NW * N,), jnp.float32),
        mesh=_mesh(),
        scratch_types=[
            pltpu.VMEM((e_per_tile,), jnp.int32),
            pltpu.VMEM((N,), jnp.float32),
        ],
        compiler_params=pltpu.CompilerParams(needs_layout_passes=False, use_tc_tiling_on_sc=False),
    )
    def k(dst_hbm, out_hbm, idx_v, deg_v):
        c = lax.axis_index("c")
        s = lax.axis_index("s")
        wid = c * NS + s

        zeros = jnp.zeros((LANES,), jnp.float32)

        def zero_body(i, _):
            deg_v[pl.ds(i * LANES, LANES)] = zeros
            return 0

        lax.fori_loop(0, N // LANES, zero_body, 0)

        pltpu.sync_copy(dst_hbm.at[pl.ds(wid * e_per_tile, e_per_tile)], idx_v)

        ones = jnp.ones((LANES,), jnp.float32)

        def body(i, _):
            idx = idx_v[pl.ds(i * LANES, LANES)]
            plsc.addupdate_scatter(deg_v, [idx], ones)
            return 0

        lax.fori_loop(0, e_per_tile // LANES, body, 0)
        pltpu.sync_copy(deg_v, out_hbm.at[pl.ds(wid * N, N)])

    return k


@functools.lru_cache(maxsize=None)
def _agg_kernel(E, N, D, CH, feat_split):
    if feat_split:
        e_per_tile = E // NS
    else:
        e_per_tile = E // NW
    n_chunks = e_per_tile // CH
    NBUF = 4
    K = 2
    assert n_chunks % NBUF == 0 and K < NBUF
    RC = 80
    n_row_chunks = N // RC
    row_iters = -(-n_row_chunks // NS)
    chunk_bytes = CH * D * 4

    out_shape = (N, NC * D) if feat_split else (NC, N, D)

    @functools.partial(
        pl.kernel,
        out_type=jax.ShapeDtypeStruct(out_shape, jnp.float32),
        mesh=_mesh(),
        scratch_types=[
            pltpu.VMEM((n_chunks, CH), jnp.int32),
            pltpu.VMEM((n_chunks, CH), jnp.int32),
            pltpu.VMEM((NBUF, CH, D), jnp.float32),
            pltpu.VMEM_SHARED((N, D), jnp.float32),
            pltpu.SemaphoreType.DMA((NBUF,)),
            pltpu.SemaphoreType.DMA((NBUF,)),
        ],
        compiler_params=pltpu.CompilerParams(needs_layout_passes=False, use_tc_tiling_on_sc=False),
    )
    def k(u_hbm, src_hbm, dst_hbm, zeros_hbm, out_hbm, si_v, di_v, rows_v,
          acc_sh, gsem, ssem):
        c = lax.axis_index("c")
        s = lax.axis_index("s")
        wid = (s if feat_split else c * NS + s)
        u_tab = u_hbm.at[c] if feat_split else u_hbm

        def zero_body(i, _):
            rc = i * NS + s

            @pl.when(rc < n_row_chunks)
            def _():
                pltpu.sync_copy(zeros_hbm, acc_sh.at[pl.ds(rc * RC, RC)])

            return 0

        lax.fori_loop(0, row_iters, zero_body, 0)

        row0 = wid * n_chunks
        pltpu.sync_copy(src_hbm.at[pl.ds(row0, n_chunks)], si_v)
        pltpu.sync_copy(dst_hbm.at[pl.ds(row0, n_chunks)], di_v)
        plsc.subcore_barrier()

        def gather(ch, b):
            pltpu.async_copy(u_tab.at[si_v.at[ch]], rows_v.at[b],
                             gsem.at[b])

        def scatter(ch, b):
            pltpu.async_copy(rows_v.at[b], acc_sh.at[di_v.at[ch]],
                             ssem.at[b], add=True)

        def drain(sem):
            pltpu.make_async_copy(u_tab.at[pl.ds(0, CH)],
                                  rows_v.at[0], sem).wait()

        for b in range(K):
            gather(b, b)

        def body(ib, _):
            for b in range(NBUF):
                ch = ib * NBUF + b
                chg = ch + K
                bg = (b + K) % NBUF

                @pl.when(chg < n_chunks)
                def _():
                    @pl.when(chg >= NBUF)
                    def _():
                        drain(ssem.at[bg])

                    gather(chg, bg)

                drain(gsem.at[b])
                scatter(ch, b)
            return 0

        lax.fori_loop(0, n_chunks // NBUF, body, 0)

        for b in range(NBUF):
            drain(ssem.at[b])

        plsc.subcore_barrier()

        def out_body(i, _):
            rc = i * NS + s

            @pl.when(rc < n_row_chunks)
            def _():
                if feat_split:
                    dst_slc = out_hbm.at[pl.ds(rc * RC, RC),
                                         pl.ds(c * D, D)]
                else:
                    dst_slc = out_hbm.at[c, pl.ds(rc * RC, RC)]
                pltpu.sync_copy(acc_sh.at[pl.ds(rc * RC, RC)], dst_slc)

            return 0

        lax.fori_loop(0, row_iters, out_body, 0)

    return k


@functools.lru_cache(maxsize=None)
def _mm_kernel(N, F, H, DP, BLK):
    grid = N // BLK

    def body(s0, u1, dinv, w1, b1, w2, o):
        agg = (s0[...] + u1[...]) * dinv[...]
        h = jnp.dot(agg, w1[...], preferred_element_type=jnp.float32)
        h = jnp.maximum(h + b1[...], 0.0)
        t = jnp.dot(h, w2[...], preferred_element_type=jnp.float32)
        o[...] = t * dinv[...]

    return pl.pallas_call(
        body,
        grid=(grid,),
        in_specs=[
            pl.BlockSpec((BLK, F), lambda i: (i, 0)),
            pl.BlockSpec((BLK, F), lambda i: (i, 0)),
            pl.BlockSpec((BLK, 1), lambda i: (i, 0)),
            pl.BlockSpec((F, H), lambda i: (0, 0)),
            pl.BlockSpec((1, H), lambda i: (0, 0)),
            pl.BlockSpec((H, DP), lambda i: (0, 0)),
        ],
        out_specs=pl.BlockSpec((BLK, DP), lambda i: (i, 0)),
        out_shape=jax.ShapeDtypeStruct((N, DP), jnp.float32),
    )


def kernel(x, edge_index, W1, b1, W2, b2):
    N, F = x.shape
    E = edge_index.shape[1]
    H = W1.shape[1]
    C = W2.shape[1]
    D2 = 16
    DP = 128

    CH = 125
    src = edge_index[0]
    dst = edge_index[1]
    src2 = src.reshape(E // CH, CH)
    dst2 = dst.reshape(E // CH, CH)

    degp = _deg_kernel(E, N)(dst).reshape(NW, N)
    deg = jnp.sum(degp, axis=0) + 1.0
    dinv = lax.rsqrt(deg)

    u1 = x * dinv[:, None]
    DH = F // NC
    u1s = u1.reshape(N, NC, DH).transpose(1, 0, 2)
    z1 = jnp.zeros((80, DH), jnp.float32)
    S1 = _agg_kernel(E, N, DH, CH, True)(u1s, src2, dst2, z1)

    W2p = jnp.zeros((H, DP), jnp.float32).at[:, :C].set(W2)
    u2p = _mm_kernel(N, F, H, DP, 2000)(
        S1, u1, dinv[:, None], W1, b1[None, :], W2p)
    u2 = u2p[:, :D2]

    z2 = jnp.zeros((80, D2), jnp.float32)
    S2 = _agg_kernel(E, N, D2, CH, False)(u2, src2, dst2, z2)

    out = (S2[0] + S2[1] + u2)[:, :C] * dinv[:, None] + b2[None, :]
    return out


# speedup vs baseline: 41.7346x; 1.0252x over previous
import functools

import jax
import jax.numpy as jnp
from jax import lax
from jax.experimental import pallas as pl
from jax.experimental.pallas import tpu as pltpu
from jax.experimental.pallas import tpu_sc as plsc

NC = 2
NS = 16
NW = NC * NS
LANES = 16


def _mesh():
    return plsc.VectorSubcoreMesh(core_axis_name="c", subcore_axis_name="s")


@functools.lru_cache(maxsize=None)
def _deg_kernel(E, N):
    e_per_tile = E // NW

    @functools.partial(
        pl.kernel,
        out_type=jax.ShapeDtypeStruct((NW * N,), jnp.float32),
        mesh=_mesh(),
        scratch_types=[
            pltpu.VMEM((e_per_tile,), jnp.int32),
            pltpu.VMEM((N,), jnp.float32),
        ],
        compiler_params=pltpu.CompilerParams(needs_layout_passes=False, use_tc_tiling_on_sc=False),
    )
    def k(dst_hbm, out_hbm, idx_v, deg_v):
        c = lax.axis_index("c")
        s = lax.axis_index("s")
        wid = c * NS + s

        zeros = jnp.zeros((LANES,), jnp.float32)

        def zero_body(i, _):
            deg_v[pl.ds(i * LANES, LANES)] = zeros
            return 0

        lax.fori_loop(0, N // LANES, zero_body, 0)

        pltpu.sync_copy(dst_hbm.at[pl.ds(wid * e_per_tile, e_per_tile)], idx_v)

        ones = jnp.ones((LANES,), jnp.float32)

        def body(i, _):
            idx = idx_v[pl.ds(i * LANES, LANES)]
            plsc.addupdate_scatter(deg_v, [idx], ones)
            return 0

        lax.fori_loop(0, e_per_tile // LANES, body, 0)
        pltpu.sync_copy(deg_v, out_hbm.at[pl.ds(wid * N, N)])

    return k


@functools.lru_cache(maxsize=None)
def _agg_kernel(E, N, D, CH, feat_split, NBUF, K, M=1):
    if feat_split:
        e_per_tile = E // NS
    else:
        e_per_tile = E // NW
    n_chunks = e_per_tile // CH
    n_groups = n_chunks // M
    assert n_chunks % M == 0 and n_groups % NBUF == 0 and K < NBUF
    RC = 80
    n_row_chunks = N // RC
    row_iters = -(-n_row_chunks // NS)
    chunk_bytes = CH * D * 4

    out_shape = (N, NC * D) if feat_split else (NC, N, D)

    @functools.partial(
        pl.kernel,
        out_type=jax.ShapeDtypeStruct(out_shape, jnp.float32),
        mesh=_mesh(),
        scratch_types=[
            pltpu.VMEM((n_chunks, CH), jnp.int32),
            pltpu.VMEM((n_chunks, CH), jnp.int32),
            pltpu.VMEM((NBUF, M, CH, D), jnp.float32),
            pltpu.VMEM_SHARED((N, D), jnp.float32),
            pltpu.SemaphoreType.DMA((NBUF,)),
            pltpu.SemaphoreType.DMA((NBUF,)),
        ],
        compiler_params=pltpu.CompilerParams(needs_layout_passes=False, use_tc_tiling_on_sc=False),
    )
    def k(u_hbm, src_hbm, dst_hbm, zeros_hbm, out_hbm, si_v, di_v, rows_v,
          acc_sh, gsem, ssem):
        c = lax.axis_index("c")
        s = lax.axis_index("s")
        wid = (s if feat_split else c * NS + s)
        u_tab = u_hbm.at[c] if feat_split else u_hbm

        def zero_body(i, _):
            rc = i * NS + s

            @pl.when(rc < n_row_chunks)
            def _():
                pltpu.sync_copy(zeros_hbm, acc_sh.at[pl.ds(rc * RC, RC)])

            return 0

        lax.fori_loop(0, row_iters, zero_body, 0)

        row0 = wid * n_chunks
        pltpu.sync_copy(src_hbm.at[pl.ds(row0, n_chunks)], si_v)
        pltpu.sync_copy(dst_hbm.at[pl.ds(row0, n_chunks)], di_v)
        plsc.subcore_barrier()

        def gather(g, b):
            for j in range(M):
                pltpu.async_copy(u_tab.at[si_v.at[g * M + j]],
                                 rows_v.at[b, j], gsem.at[b])

        def scatter(g, b):
            for j in range(M):
                pltpu.async_copy(rows_v.at[b, j],
                                 acc_sh.at[di_v.at[g * M + j]],
                                 ssem.at[b], add=True)

        def drain(sem):
            for j in range(M):
                pltpu.make_async_copy(u_tab.at[pl.ds(0, CH)],
                                      rows_v.at[0, j], sem).wait()

        for b in range(K):
            gather(b, b)

        def body(ib, _):
            for b in range(NBUF):
                g = ib * NBUF + b
                gg = g + K
                bg = (b + K) % NBUF

                @pl.when(gg < n_groups)
                def _():
                    @pl.when(gg >= NBUF)
                    def _():
                        drain(ssem.at[bg])

                    gather(gg, bg)

                drain(gsem.at[b])
                scatter(g, b)
            return 0

        lax.fori_loop(0, n_groups // NBUF, body, 0)

        for b in range(NBUF):
            drain(ssem.at[b])

        plsc.subcore_barrier()

        def out_body(i, _):
            rc = i * NS + s

            @pl.when(rc < n_row_chunks)
            def _():
                if feat_split:
                    dst_slc = out_hbm.at[pl.ds(rc * RC, RC),
                                         pl.ds(c * D, D)]
                else:
                    dst_slc = out_hbm.at[c, pl.ds(rc * RC, RC)]
                pltpu.sync_copy(acc_sh.at[pl.ds(rc * RC, RC)], dst_slc)

            return 0

        lax.fori_loop(0, row_iters, out_body, 0)

    return k


@functools.lru_cache(maxsize=None)
def _mm_kernel(N, F, H, DP, BLK):
    grid = N // BLK
    DH = F // NC

    def body(s0, ua, ub, dinv, w1, b1, w2, o):
        u = jnp.concatenate([ua[0], ub[0]], axis=1)
        agg = (s0[...] + u) * dinv[...]
        h = jnp.dot(agg, w1[...], preferred_element_type=jnp.float32)
        h = jnp.maximum(h + b1[...], 0.0)
        t = jnp.dot(h, w2[...], preferred_element_type=jnp.float32)
        o[...] = t * dinv[...]

    return pl.pallas_call(
        body,
        grid=(grid,),
        in_specs=[
            pl.BlockSpec((BLK, F), lambda i: (i, 0)),
            pl.BlockSpec((1, BLK, DH), lambda i: (0, i, 0)),
            pl.BlockSpec((1, BLK, DH), lambda i: (1, i, 0)),
            pl.BlockSpec((BLK, 1), lambda i: (i, 0)),
            pl.BlockSpec((F, H), lambda i: (0, 0)),
            pl.BlockSpec((1, H), lambda i: (0, 0)),
            pl.BlockSpec((H, DP), lambda i: (0, 0)),
        ],
        out_specs=pl.BlockSpec((BLK, DP), lambda i: (i, 0)),
        out_shape=jax.ShapeDtypeStruct((N, DP), jnp.float32),
    )


def kernel(x, edge_index, W1, b1, W2, b2):
    N, F = x.shape
    E = edge_index.shape[1]
    H = W1.shape[1]
    C = W2.shape[1]
    D2 = 16
    DP = 128

    CH = 125
    src = edge_index[0]
    dst = edge_index[1]
    src2 = src.reshape(E // CH, CH)
    dst2 = dst.reshape(E // CH, CH)

    degp = _deg_kernel(E, N)(dst).reshape(NW, N)
    deg = jnp.sum(degp, axis=0) + 1.0
    dinv = lax.rsqrt(deg)

    DH = F // NC
    u1s = (x * dinv[:, None]).reshape(N, NC, DH).transpose(1, 0, 2)
    z1 = jnp.zeros((80, DH), jnp.float32)
    S1 = _agg_kernel(E, N, DH, CH, True, 5, 2, 1)(u1s, src2, dst2, z1)

    W2p = jnp.zeros((H, D2), jnp.float32).at[:, :C].set(W2)
    u2 = _mm_kernel(N, F, H, D2, 2000)(
        S1, u1s, u1s, dinv[:, None], W1, b1[None, :], W2p)

    z2 = jnp.zeros((80, D2), jnp.float32)
    S2 = _agg_kernel(E, N, D2, CH, False, 4, 2, 5)(u2, src2, dst2, z2)

    out = (S2[0] + S2[1] + u2)[:, :C] * dinv[:, None] + b2[None, :]
    return out


# speedup vs baseline: 47.0431x; 1.1272x over previous
import functools

import jax
import jax.numpy as jnp
from jax import lax
from jax.experimental import pallas as pl
from jax.experimental.pallas import tpu as pltpu
from jax.experimental.pallas import tpu_sc as plsc

NC = 2
NS = 16
NW = NC * NS
LANES = 16


def _mesh():
    return plsc.VectorSubcoreMesh(core_axis_name="c", subcore_axis_name="s")


@functools.lru_cache(maxsize=None)
def _deg_kernel(E, N):
    e_per_tile = E // NW

    @functools.partial(
        pl.kernel,
        out_type=jax.ShapeDtypeStruct((NW * N,), jnp.float32),
        mesh=_mesh(),
        scratch_types=[
            pltpu.VMEM((e_per_tile,), jnp.int32),
            pltpu.VMEM((N,), jnp.float32),
        ],
        compiler_params=pltpu.CompilerParams(needs_layout_passes=False, use_tc_tiling_on_sc=False),
    )
    def k(ei_hbm, out_hbm, idx_v, deg_v):
        c = lax.axis_index("c")
        s = lax.axis_index("s")
        wid = c * NS + s

        zeros = jnp.zeros((LANES,), jnp.float32)

        def zero_body(i, _):
            deg_v[pl.ds(i * LANES, LANES)] = zeros
            return 0

        lax.fori_loop(0, N // LANES, zero_body, 0)

        pltpu.sync_copy(ei_hbm.at[1, pl.ds(wid * e_per_tile, e_per_tile)],
                        idx_v)

        ones = jnp.ones((LANES,), jnp.float32)

        def body(i, _):
            idx = idx_v[pl.ds(i * LANES, LANES)]
            plsc.addupdate_scatter(deg_v, [idx], ones)
            return 0

        lax.fori_loop(0, e_per_tile // LANES, body, 0)
        pltpu.sync_copy(deg_v, out_hbm.at[pl.ds(wid * N, N)])

    return k


@functools.lru_cache(maxsize=None)
def _agg_kernel(E, N, D, CH, feat_split, NBUF, K, M=1):
    if feat_split:
        e_per_tile = E // NS
    else:
        e_per_tile = E // NW
    n_chunks = e_per_tile // CH
    n_groups = n_chunks // M
    assert n_chunks % M == 0 and n_groups % NBUF == 0 and K < NBUF
    RC = 80
    n_row_chunks = N // RC
    row_iters = -(-n_row_chunks // NS)
    chunk_bytes = CH * D * 4

    out_shape = (N, NC * D) if feat_split else (NC, N, D)

    @functools.partial(
        pl.kernel,
        out_type=jax.ShapeDtypeStruct(out_shape, jnp.float32),
        mesh=_mesh(),
        scratch_types=[
            pltpu.VMEM((n_chunks, CH), jnp.int32),
            pltpu.VMEM((n_chunks, CH), jnp.int32),
            pltpu.VMEM((NBUF, M, CH, D), jnp.float32),
            pltpu.VMEM_SHARED((N, D), jnp.float32),
            pltpu.SemaphoreType.DMA((NBUF,)),
            pltpu.SemaphoreType.DMA((NBUF,)),
        ],
        compiler_params=pltpu.CompilerParams(needs_layout_passes=False, use_tc_tiling_on_sc=False),
    )
    def k(u_hbm, ei_hbm, zeros_hbm, out_hbm, si_v, di_v, rows_v,
          acc_sh, gsem, ssem):
        c = lax.axis_index("c")
        s = lax.axis_index("s")
        wid = (s if feat_split else c * NS + s)
        u_tab = u_hbm.at[c] if feat_split else u_hbm

        def zero_body(i, _):
            rc = i * NS + s

            @pl.when(rc < n_row_chunks)
            def _():
                pltpu.sync_copy(zeros_hbm, acc_sh.at[pl.ds(rc * RC, RC)])

            return 0

        lax.fori_loop(0, row_iters, zero_body, 0)

        row0 = wid * n_chunks
        pltpu.sync_copy(ei_hbm.at[0, pl.ds(row0, n_chunks)], si_v)
        pltpu.sync_copy(ei_hbm.at[1, pl.ds(row0, n_chunks)], di_v)
        plsc.subcore_barrier()

        def gather(g, b):
            for j in range(M):
                pltpu.async_copy(u_tab.at[si_v.at[g * M + j]],
                                 rows_v.at[b, j], gsem.at[b])

        def scatter(g, b):
            for j in range(M):
                pltpu.async_copy(rows_v.at[b, j],
                                 acc_sh.at[di_v.at[g * M + j]],
                                 ssem.at[b], add=True)

        def drain(sem):
            for j in range(M):
                pltpu.make_async_copy(u_tab.at[pl.ds(0, CH)],
                                      rows_v.at[0, j], sem).wait()

        for b in range(K):
            gather(b, b)

        def body(ib, _):
            for b in range(NBUF):
                g = ib * NBUF + b
                gg = g + K
                bg = (b + K) % NBUF

                @pl.when(gg < n_groups)
                def _():
                    @pl.when(gg >= NBUF)
                    def _():
                        drain(ssem.at[bg])

                    gather(gg, bg)

                drain(gsem.at[b])
                scatter(g, b)
            return 0

        lax.fori_loop(0, n_groups // NBUF, body, 0)

        for b in range(NBUF):
            drain(ssem.at[b])

        plsc.subcore_barrier()

        def out_body(i, _):
            rc = i * NS + s

            @pl.when(rc < n_row_chunks)
            def _():
                if feat_split:
                    dst_slc = out_hbm.at[pl.ds(rc * RC, RC),
                                         pl.ds(c * D, D)]
                else:
                    dst_slc = out_hbm.at[c, pl.ds(rc * RC, RC)]
                pltpu.sync_copy(acc_sh.at[pl.ds(rc * RC, RC)], dst_slc)

            return 0

        lax.fori_loop(0, row_iters, out_body, 0)

    return k


@functools.lru_cache(maxsize=None)
def _mm_kernel(N, F, H, C, DP, BLK):
    grid = N // BLK

    def body(s0, ua, ub, dinv, w1, b1, w2, o):
        u1 = jnp.concatenate([ua[0], ub[0]], axis=1)
        agg = (s0[...] + u1) * dinv[...]
        h = jnp.dot(agg, w1[...], preferred_element_type=jnp.float32)
        h = jnp.maximum(h + b1[...], 0.0)
        t = jnp.dot(h, w2[...], preferred_element_type=jnp.float32)
        t = t * dinv[...]
        o[...] = jnp.pad(t, ((0, 0), (0, DP - t.shape[1])))

    return pl.pallas_call(
        body,
        grid=(grid,),
        in_specs=[
            pl.BlockSpec((BLK, F), lambda i: (i, 0)),
            pl.BlockSpec((1, BLK, F // NC), lambda i: (0, i, 0)),
            pl.BlockSpec((1, BLK, F // NC), lambda i: (1, i, 0)),
            pl.BlockSpec((BLK, 1), lambda i: (i, 0)),
            pl.BlockSpec((F, H), lambda i: (0, 0)),
            pl.BlockSpec((1, H), lambda i: (0, 0)),
            pl.BlockSpec((H, C), lambda i: (0, 0)),
        ],
        out_specs=pl.BlockSpec((BLK, DP), lambda i: (i, 0)),
        out_shape=jax.ShapeDtypeStruct((N, DP), jnp.float32),
    )


@functools.lru_cache(maxsize=None)
def _pre_kernel(N, DH, BLK):
    def body(xb, dinv, o):
        u = xb[...] * dinv[...]
        for h in range(NC):
            o[h] = u[:, h * DH:(h + 1) * DH]

    return pl.pallas_call(
        body,
        grid=(N // BLK,),
        in_specs=[
            pl.BlockSpec((BLK, NC * DH), lambda i: (i, 0)),
            pl.BlockSpec((BLK, 1), lambda i: (i, 0)),
        ],
        out_specs=pl.BlockSpec((NC, BLK, DH), lambda i: (0, i, 0)),
        out_shape=jax.ShapeDtypeStruct((NC, N, DH), jnp.float32),
    )


@functools.lru_cache(maxsize=None)
def _fin_kernel(N, C, D2, BLK):
    grid = N // BLK

    def body(sa, sb, u2, dinv, b2, o):
        t = (sa[0] + sb[0] + u2[...]) * dinv[...]
        o[...] = t[:, :C] + b2[...]

    return pl.pallas_call(
        body,
        grid=(grid,),
        in_specs=[
            pl.BlockSpec((1, BLK, D2), lambda i: (0, i, 0)),
            pl.BlockSpec((1, BLK, D2), lambda i: (1, i, 0)),
            pl.BlockSpec((BLK, D2), lambda i: (i, 0)),
            pl.BlockSpec((BLK, 1), lambda i: (i, 0)),
            pl.BlockSpec((1, C), lambda i: (0, 0)),
        ],
        out_specs=pl.BlockSpec((BLK, C), lambda i: (i, 0)),
        out_shape=jax.ShapeDtypeStruct((N, C), jnp.float32),
    )


def kernel(x, edge_index, W1, b1, W2, b2):
    N, F = x.shape
    E = edge_index.shape[1]
    H = W1.shape[1]
    C = W2.shape[1]
    D2 = 16
    DP = 128

    CH = 125
    ei3 = edge_index.reshape(2, E // CH, CH)

    degp = _deg_kernel(E, N)(edge_index).reshape(NW, N)
    deg = jnp.sum(degp, axis=0) + 1.0
    dinv = lax.rsqrt(deg)

    DH = F // NC
    u1s = _pre_kernel(N, DH, 2000)(x, dinv[:, None])
    z1 = jnp.zeros((80, DH), jnp.float32)
    S1 = _agg_kernel(E, N, DH, CH, True, 5, 2, 1)(u1s, ei3, z1)

    u2 = _mm_kernel(N, F, H, C, D2, 2000)(
        S1, u1s, u1s, dinv[:, None], W1, b1[None, :], W2)

    z2 = jnp.zeros((80, D2), jnp.float32)
    S2 = _agg_kernel(E, N, D2, CH, False, 4, 2, 5)(u2, ei3, z2)

    return _fin_kernel(N, C, D2, 2000)(S2, S2, u2, dinv[:, None], b2[None, :])


# speedup vs baseline: 48.3666x; 1.0281x over previous
import functools

import jax
import jax.numpy as jnp
from jax import lax
from jax.experimental import pallas as pl
from jax.experimental.pallas import tpu as pltpu
from jax.experimental.pallas import tpu_sc as plsc

NC = 2
NS = 16
NW = NC * NS
LANES = 16


def _mesh():
    return plsc.VectorSubcoreMesh(core_axis_name="c", subcore_axis_name="s")


@functools.lru_cache(maxsize=None)
def _deg_kernel(E, N):
    e_per_tile = E // NW

    @functools.partial(
        pl.kernel,
        out_type=jax.ShapeDtypeStruct((NW * N,), jnp.float32),
        mesh=_mesh(),
        scratch_types=[
            pltpu.VMEM((e_per_tile,), jnp.int32),
            pltpu.VMEM((N,), jnp.float32),
        ],
        compiler_params=pltpu.CompilerParams(needs_layout_passes=False, use_tc_tiling_on_sc=False),
    )
    def k(ei_hbm, out_hbm, idx_v, deg_v):
        c = lax.axis_index("c")
        s = lax.axis_index("s")
        wid = c * NS + s

        zeros = jnp.zeros((LANES,), jnp.float32)

        def zero_body(i, _):
            deg_v[pl.ds(i * LANES, LANES)] = zeros
            return 0

        lax.fori_loop(0, N // LANES, zero_body, 0)

        pltpu.sync_copy(ei_hbm.at[1, pl.ds(wid * e_per_tile, e_per_tile)],
                        idx_v)

        ones = jnp.ones((LANES,), jnp.float32)

        def body(i, _):
            idx = idx_v[pl.ds(i * LANES, LANES)]
            plsc.addupdate_scatter(deg_v, [idx], ones)
            return 0

        lax.fori_loop(0, e_per_tile // LANES, body, 0)
        pltpu.sync_copy(deg_v, out_hbm.at[pl.ds(wid * N, N)])

    return k


@functools.lru_cache(maxsize=None)
def _agg_kernel(E, N, D, CH, feat_split, NBUF, K, M=1):
    if feat_split:
        e_per_tile = E // NS
    else:
        e_per_tile = E // NW
    n_chunks = e_per_tile // CH
    n_groups = n_chunks // M
    assert n_chunks % M == 0 and n_groups % NBUF == 0 and K < NBUF
    RC = 80
    n_row_chunks = N // RC
    row_iters = -(-n_row_chunks // NS)
    chunk_bytes = CH * D * 4

    out_shape = (N, NC * D) if feat_split else (NC, N, 128)

    @functools.partial(
        pl.kernel,
        out_type=jax.ShapeDtypeStruct(out_shape, jnp.float32),
        mesh=_mesh(),
        scratch_types=[
            pltpu.VMEM((n_chunks, CH), jnp.int32),
            pltpu.VMEM((n_chunks, CH), jnp.int32),
            pltpu.VMEM((NBUF, M, CH, D), jnp.float32),
            pltpu.VMEM_SHARED((N, D), jnp.float32),
            pltpu.SemaphoreType.DMA((NBUF,)),
            pltpu.SemaphoreType.DMA((NBUF,)),
        ],
        compiler_params=pltpu.CompilerParams(needs_layout_passes=False, use_tc_tiling_on_sc=False),
    )
    def k(u_hbm, ei_hbm, zeros_hbm, out_hbm, si_v, di_v, rows_v,
          acc_sh, gsem, ssem):
        c = lax.axis_index("c")
        s = lax.axis_index("s")
        wid = (s if feat_split else c * NS + s)
        u_tab = u_hbm.at[c] if feat_split else u_hbm

        def zero_body(i, _):
            rc = i * NS + s

            @pl.when(rc < n_row_chunks)
            def _():
                pltpu.sync_copy(zeros_hbm, acc_sh.at[pl.ds(rc * RC, RC)])

            return 0

        lax.fori_loop(0, row_iters, zero_body, 0)

        row0 = wid * n_chunks
        pltpu.sync_copy(ei_hbm.at[0, pl.ds(row0, n_chunks)], si_v)
        pltpu.sync_copy(ei_hbm.at[1, pl.ds(row0, n_chunks)], di_v)
        plsc.subcore_barrier()

        def gather(g, b):
            for j in range(M):
                pltpu.async_copy(u_tab.at[si_v.at[g * M + j]],
                                 rows_v.at[b, j], gsem.at[b])

        def scatter(g, b):
            for j in range(M):
                pltpu.async_copy(rows_v.at[b, j],
                                 acc_sh.at[di_v.at[g * M + j]],
                                 ssem.at[b], add=True)

        def drain(sem):
            for j in range(M):
                pltpu.make_async_copy(u_tab.at[pl.ds(0, CH)],
                                      rows_v.at[0, j], sem).wait()

        for b in range(K):
            gather(b, b)

        def body(ib, _):
            for b in range(NBUF):
                g = ib * NBUF + b
                gg = g + K
                bg = (b + K) % NBUF

                @pl.when(gg < n_groups)
                def _():
                    @pl.when(gg >= NBUF)
                    def _():
                        drain(ssem.at[bg])

                    gather(gg, bg)

                drain(gsem.at[b])
                scatter(g, b)
            return 0

        lax.fori_loop(0, n_groups // NBUF, body, 0)

        for b in range(NBUF):
            drain(ssem.at[b])

        plsc.subcore_barrier()

        def out_body(i, _):
            rc = i * NS + s

            @pl.when(rc < n_row_chunks)
            def _():
                if feat_split:
                    dst_slc = out_hbm.at[pl.ds(rc * RC, RC),
                                         pl.ds(c * D, D)]
                else:
                    dst_slc = out_hbm.at[c, pl.ds(rc * RC, RC),
                                         pl.ds(0, D)]
                pltpu.sync_copy(acc_sh.at[pl.ds(rc * RC, RC)], dst_slc)

            return 0

        lax.fori_loop(0, row_iters, out_body, 0)

    return k


@functools.lru_cache(maxsize=None)
def _agg128_kernel(E, N, D, CH):
    e_per_tile = E // NW
    n_chunks = e_per_tile // CH
    NBUF = 2
    BLKC = 8
    n_blocks = n_chunks // BLKC
    assert n_chunks % BLKC == 0
    RC = 80
    n_row_chunks = N // RC
    row_iters = -(-n_row_chunks // NS)

    @functools.partial(
        pl.kernel,
        out_type=jax.ShapeDtypeStruct((NC, N, D), jnp.float32),
        mesh=_mesh(),
        scratch_types=[
            pltpu.VMEM((n_chunks, CH), jnp.int32),
            pltpu.VMEM((2, BLKC, CH), jnp.int32),
            pltpu.VMEM((NBUF, CH, D), jnp.float32),
            pltpu.VMEM_SHARED((N, D), jnp.float32),
            pltpu.SemaphoreType.DMA((NBUF,)),
            pltpu.SemaphoreType.DMA((NBUF,)),
            pltpu.SemaphoreType.DMA((2,)),
        ],
        compiler_params=pltpu.CompilerParams(needs_layout_passes=False, use_tc_tiling_on_sc=False),
    )
    def k(u_hbm, ei_hbm, zeros_hbm, out_hbm, si_v, dr_v, rows_v,
          acc_sh, gsem, ssem, isem):
        c = lax.axis_index("c")
        s = lax.axis_index("s")
        wid = c * NS + s
        row0 = wid * n_chunks

        def zero_body(i, _):
            rc = i * NS + s

            @pl.when(rc < n_row_chunks)
            def _():
                pltpu.sync_copy(zeros_hbm, acc_sh.at[pl.ds(rc * RC, RC)])

            return 0

        lax.fori_loop(0, row_iters, zero_body, 0)
        pltpu.sync_copy(ei_hbm.at[0, pl.ds(row0, n_chunks)], si_v)
        plsc.subcore_barrier()

        def fetch_di( blk, slot):
            pltpu.async_copy(
                ei_hbm.at[1, pl.ds(row0 + blk * BLKC, BLKC)],
                dr_v.at[slot], isem.at[slot])

        def gather(ch, b):
            pltpu.async_copy(u_hbm.at[si_v.at[ch]], rows_v.at[b],
                             gsem.at[b])

        def scatter(slot, j, b):
            pltpu.async_copy(rows_v.at[b], acc_sh.at[dr_v.at[slot, j]],
                             ssem.at[b], add=True)

        def drain_rows(sem):
            pltpu.make_async_copy(u_hbm.at[pl.ds(0, CH)],
                                  rows_v.at[0], sem).wait()

        def drain_di(sem):
            pltpu.make_async_copy(ei_hbm.at[1, pl.ds(0, BLKC)],
                                  dr_v.at[0], sem).wait()

        fetch_di(0, 0)
        gather(0, 0)

        def body(ib, _):
            slot = lax.rem(ib, 2)
            for b in range(BLKC):
                ch = ib * BLKC + b
                if b == 0:
                    drain_di(isem.at[slot])

                    @pl.when(ib + 1 < n_blocks)
                    def _():
                        fetch_di(ib + 1, 1 - slot)

                @pl.when(ch + 1 < n_chunks)
                def _():
                    @pl.when(ch + 1 >= NBUF)
                    def _():
                        drain_rows(ssem.at[(b + 1) % NBUF])

                    gather(ch + 1, (b + 1) % NBUF)

                drain_rows(gsem.at[b % NBUF])
                scatter(slot, b, b % NBUF)
            return 0

        lax.fori_loop(0, n_blocks, body, 0)
        for b in range(NBUF):
            drain_rows(ssem.at[b])

        plsc.subcore_barrier()

        def out_body(i, _):
            rc = i * NS + s

            @pl.when(rc < n_row_chunks)
            def _():
                pltpu.sync_copy(acc_sh.at[pl.ds(rc * RC, RC)],
                                out_hbm.at[c, pl.ds(rc * RC, RC)])

            return 0

        lax.fori_loop(0, row_iters, out_body, 0)

    return k


@functools.lru_cache(maxsize=None)
def _mm_kernel(N, F, H, C, DP, BLK):
    grid = N // BLK

    def body(sa, sb, u1, dinv, w1, b1, w2, o):
        agg = (sa[0] + sb[0] + u1[...]) * dinv[...]
        h = jnp.dot(agg, w1[...], preferred_element_type=jnp.float32)
        h = jnp.maximum(h + b1[...], 0.0)
        t = jnp.dot(h, w2[...], preferred_element_type=jnp.float32)
        t = t * dinv[...]
        o[...] = jnp.pad(t, ((0, 0), (0, DP - t.shape[1])))

    return pl.pallas_call(
        body,
        grid=(grid,),
        in_specs=[
            pl.BlockSpec((1, BLK, F), lambda i: (0, i, 0)),
            pl.BlockSpec((1, BLK, F), lambda i: (1, i, 0)),
            pl.BlockSpec((BLK, F), lambda i: (i, 0)),
            pl.BlockSpec((BLK, 1), lambda i: (i, 0)),
            pl.BlockSpec((F, H), lambda i: (0, 0)),
            pl.BlockSpec((1, H), lambda i: (0, 0)),
            pl.BlockSpec((H, C), lambda i: (0, 0)),
        ],
        out_specs=pl.BlockSpec((BLK, DP), lambda i: (i, 0)),
        out_shape=jax.ShapeDtypeStruct((N, DP), jnp.float32),
    )


@functools.lru_cache(maxsize=None)
def _pre_kernel(N, DH, BLK):
    def body(xb, dinv, o):
        u = xb[...] * dinv[...]
        for h in range(NC):
            o[h] = u[:, h * DH:(h + 1) * DH]

    return pl.pallas_call(
        body,
        grid=(N // BLK,),
        in_specs=[
            pl.BlockSpec((BLK, NC * DH), lambda i: (i, 0)),
            pl.BlockSpec((BLK, 1), lambda i: (i, 0)),
        ],
        out_specs=pl.BlockSpec((NC, BLK, DH), lambda i: (0, i, 0)),
        out_shape=jax.ShapeDtypeStruct((NC, N, DH), jnp.float32),
    )


@functools.lru_cache(maxsize=None)
def _fin_kernel(N, C, D2, BLK):
    grid = N // BLK

    def body(sa, sb, u2, dinv, b2, o):
        t = (sa[0, :, :D2] + sb[0, :, :D2] + u2[...]) * dinv[...]
        o[...] = t[:, :C] + b2[...]

    return pl.pallas_call(
        body,
        grid=(grid,),
        in_specs=[
            pl.BlockSpec((1, BLK, 128), lambda i: (0, i, 0)),
            pl.BlockSpec((1, BLK, 128), lambda i: (1, i, 0)),
            pl.BlockSpec((BLK, D2), lambda i: (i, 0)),
            pl.BlockSpec((BLK, 1), lambda i: (i, 0)),
            pl.BlockSpec((1, C), lambda i: (0, 0)),
        ],
        out_specs=pl.BlockSpec((BLK, C), lambda i: (i, 0)),
        out_shape=jax.ShapeDtypeStruct((N, C), jnp.float32),
    )


def kernel(x, edge_index, W1, b1, W2, b2):
    N, F = x.shape
    E = edge_index.shape[1]
    H = W1.shape[1]
    C = W2.shape[1]
    D2 = 16
    DP = 128

    CH = 125
    ei3 = edge_index.reshape(2, E // CH, CH)

    degp = _deg_kernel(E, N)(edge_index).reshape(NW, N)
    deg = jnp.sum(degp, axis=0) + 1.0
    dinv = lax.rsqrt(deg)

    u1 = x * dinv[:, None]
    z1 = jnp.zeros((80, F), jnp.float32)
    S1 = _agg128_kernel(E, N, F, CH)(u1, ei3, z1)

    u2 = _mm_kernel(N, F, H, C, D2, 2000)(
        S1, S1, u1, dinv[:, None], W1, b1[None, :], W2)

    z2 = jnp.zeros((80, D2), jnp.float32)
    S2 = _agg_kernel(E, N, D2, CH, False, 4, 2, 5)(u2, ei3, z2)

    return _fin_kernel(N, C, D2, 2000)(S2, S2, u2, dinv[:, None], b2[None, :])
